# baseline (speedup 1.0000x reference)
import functools

import jax
import jax.numpy as jnp
from jax import lax
from jax.experimental import pallas as pl
from jax.experimental.pallas import tpu as pltpu
from jax.experimental.pallas import tpu_sc as plsc

NC, NS, L = 2, 16, 16
NW = NC * NS
NPAIR = 5


def _make_sc_sampler(B, NR, P, D, H, W, CF):
    N = B * NR * P
    NRAYS = B * NR
    RPT = NRAYS // NW
    PPT = RPT * P
    CP = 64
    GROUPS = CP // L
    RAYS_PER_CHUNK = CP // P
    GROUPS_PER_RAY = P // L
    CHUNKS = PPT // CP
    CC = 1 + CF
    DHW = D * H * W
    DHW8 = DHW // 8
    NSLOT = 8 * NPAIR
    assert NRAYS % NW == 0 and P % L == 0 and CP % P == 0 and PPT % CP == 0
    assert CF == 8 and W % 8 == 0

    mesh = plsc.VectorSubcoreMesh(core_axis_name="c", subcore_axis_name="s")

    @functools.partial(
        pl.kernel,
        mesh=mesh,
        compiler_params=pltpu.CompilerParams(
            needs_layout_passes=False, use_tc_tiling_on_sc=False),
        out_type=(
            jax.ShapeDtypeStruct((N,), jnp.float32),
            jax.ShapeDtypeStruct((N, CF), jnp.float32),
        ),
        scratch_types=[
            pltpu.VMEM((RPT,), jnp.float32),
            pltpu.VMEM((RPT,), jnp.float32),
            pltpu.VMEM((RPT,), jnp.float32),
            pltpu.VMEM((RPT,), jnp.float32),
            pltpu.VMEM((RPT,), jnp.float32),
            pltpu.VMEM((RPT,), jnp.float32),
            pltpu.VMEM((CP,), jnp.float32),
            pltpu.VMEM((NSLOT, CP), jnp.int32),
            pltpu.VMEM((8, GROUPS, L), jnp.float32),
            pltpu.VMEM((2, GROUPS, L), jnp.int32),
            pltpu.VMEM((NSLOT, CP, 8), jnp.int32),
            pltpu.VMEM((CP,), jnp.float32),
            pltpu.VMEM((CP, CF), jnp.float32),
            pltpu.SemaphoreType.DMA,
        ],
    )
    def sampler(pairs_h, ox_h, oy_h, oz_h, dx_h, dy_h, dz_h, t_h,
                dens_h, feat_h,
                oxv, oyv, ozv, dxv, dyv, dzv, tv, idxv, wv, ov, rowsv,
                densv, featv, gsem):
        cid = lax.axis_index("c")
        sid = lax.axis_index("s")
        wid = sid * NC + cid
        ray_base = wid * RPT
        pt_base = wid * PPT
        pair_base = (ray_base // NR) * (NPAIR * DHW8)

        pltpu.sync_copy(ox_h.at[pl.ds(ray_base, RPT)], oxv)
        pltpu.sync_copy(oy_h.at[pl.ds(ray_base, RPT)], oyv)
        pltpu.sync_copy(oz_h.at[pl.ds(ray_base, RPT)], ozv)
        pltpu.sync_copy(dx_h.at[pl.ds(ray_base, RPT)], dxv)
        pltpu.sync_copy(dy_h.at[pl.ds(ray_base, RPT)], dyv)
        pltpu.sync_copy(dz_h.at[pl.ds(ray_base, RPT)], dzv)

        iota = lax.iota(jnp.int32, L)
        fone = jnp.full((L,), 1.0, jnp.float32)
        fzero = jnp.full((L,), 0.0, jnp.float32)

        def axis_setup(pval, extent):
            gc = (pval + 1.0) * (0.5 * (extent - 1))
            ti = gc.astype(jnp.int32)
            tf = ti.astype(jnp.float32)
            neg = (gc < tf)
            lo = ti - neg.astype(jnp.int32)
            lof = tf - neg.astype(jnp.float32)
            fr = gc - lof
            w_lo = fone - fr
            w_hi = fr
            v_lo = (lo >= 0) & (lo <= extent - 1)
            v_hi = (lo >= -1) & (lo <= extent - 2)
            w_lo = jnp.where(v_lo, w_lo, fzero)
            w_hi = jnp.where(v_hi, w_hi, fzero)
            lo_c = jnp.clip(lo, 0, extent - 1)
            hi_c = jnp.clip(lo + 1, 0, extent - 1)
            return w_lo, w_hi, lo_c, hi_c

        def chunk_body(ci, carry):
            pltpu.sync_copy(t_h.at[pl.ds(pt_base + ci * CP, CP)], tv)
            for g in range(GROUPS):
                ray_l = ci * RAYS_PER_CHUNK + (g // GROUPS_PER_RAY)
                ridx = jnp.full((L,), ray_l, jnp.int32)
                oxs = plsc.load_gather(oxv, [ridx])
                oys = plsc.load_gather(oyv, [ridx])
                ozs = plsc.load_gather(ozv, [ridx])
                dxs = plsc.load_gather(dxv, [ridx])
                dys = plsc.load_gather(dyv, [ridx])
                dzs = plsc.load_gather(dzv, [ridx])
                t16 = tv[pl.ds(g * L, L)]
                px = oxs + dxs * t16
                py = oys + dys * t16
                pz = ozs + dzs * t16
                wx0, wx1, x0, x1 = axis_setup(px, W)
                wy0, wy1, y0, y1 = axis_setup(py, H)
                wz0, wz1, z0, z1 = axis_setup(pz, D)
                xr0 = jnp.right_shift(x0, 3)
                xr1 = jnp.right_shift(x1, 3)
                ov[0, g, :] = jnp.bitwise_and(x0, 7)
                ov[1, g, :] = jnp.bitwise_and(x1, 7)
                zy = (
                    z0 * (H * W // 8) + y0 * (W // 8),
                    z0 * (H * W // 8) + y1 * (W // 8),
                    z1 * (H * W // 8) + y0 * (W // 8),
                    z1 * (H * W // 8) + y1 * (W // 8),
                )
                wzy = (wy0 * wz0, wy1 * wz0, wy0 * wz1, wy1 * wz1)
                for j in range(4):
                    for xi, (xr, wx) in enumerate(((xr0, wx0), (xr1, wx1))):
                        kw = j * 2 + xi
                        wv[kw, g, :] = wzy[j] * wx
                        base = zy[j] + xr + pair_base
                        for w in range(NPAIR):
                            idxv[kw * NPAIR + w, pl.ds(g * L, L)] = (
                                base + w * DHW8)
            handles = [
                pltpu.async_copy(pairs_h.at[idxv.at[s]], rowsv.at[s], gsem)
                for s in range(NSLOT)
            ]
            for h in handles:
                h.wait()
            for g in range(GROUPS):
                p_idx = iota + (g * L)
                accs = [None] * (CC + 1)

                def acc_add(c, contrib):
                    accs[c] = contrib if accs[c] is None else accs[c] + contrib

                o0v = ov[0, g, :]
                o1v = ov[1, g, :]
                for kw in range(8):
                    wk = wv[kw, g, :]
                    ovv = o1v if (kw & 1) else o0v
                    for w in range(NPAIR):
                        slot = kw * NPAIR + w
                        svec = jnp.full((L,), slot, jnp.int32)
                        val = plsc.load_gather(rowsv, [svec, p_idx, ovv])
                        vb = plsc.bitcast(val, jnp.bfloat16)
                        alo, ahi = plsc.unpack(
                            vb, format=plsc.PackFormat.INTERLEAVED)
                        acc_add(2 * w, wk * alo)
                        if 2 * w + 1 < CC:
                            acc_add(2 * w + 1, wk * ahi)
                densv[pl.ds(g * L, L)] = accs[0]
                for c in range(1, CC):
                    plsc.store_scatter(
                        featv, [p_idx, jnp.full((L,), c - 1, jnp.int32)],
                        accs[c])
            pltpu.sync_copy(densv, dens_h.at[pl.ds(pt_base + ci * CP, CP)])
            pltpu.sync_copy(featv, feat_h.at[pl.ds(pt_base + ci * CP, CP)])
            return carry

        lax.fori_loop(0, CHUNKS, chunk_body, 0)

    return sampler


def _bf16_bits(x):
    u = lax.bitcast_convert_type(x, jnp.uint32)
    return (u + 0x7FFF + (jnp.right_shift(u, 16) & 1)) >> 16


def _pack_pair(lo, hi):
    word = _bf16_bits(lo) | (_bf16_bits(hi) << 16)
    return lax.bitcast_convert_type(word, jnp.int32)


def kernel(origins, directions, lengths, densities, features, world2local):
    B, NR, _ = origins.shape
    P = lengths.shape[-1]
    _, CD, D, H, W = densities.shape
    CF = features.shape[1]
    DHW = D * H * W

    ones = jnp.ones(origins.shape[:-1] + (1,), dtype=origins.dtype)
    o_h = jnp.concatenate([origins, ones], axis=-1)
    o_loc = jnp.einsum('bnk,bkj->bnj', o_h, world2local)
    o_loc = o_loc[..., :3] / o_loc[..., 3:4]
    d_loc = jnp.einsum('bnk,bkj->bnj', directions, world2local[:, :3, :3])

    ox = o_loc[..., 0].reshape(-1)
    oy = o_loc[..., 1].reshape(-1)
    oz = o_loc[..., 2].reshape(-1)
    dx = d_loc[..., 0].reshape(-1)
    dy = d_loc[..., 1].reshape(-1)
    dz = d_loc[..., 2].reshape(-1)
    tflat = lengths.reshape(-1)

    dens2 = densities.reshape(B, DHW)
    feat2 = features.reshape(B, CF, DHW)
    zero = jnp.zeros_like(dens2)
    pairs = jnp.stack([
        _pack_pair(dens2, feat2[:, 0]),
        _pack_pair(feat2[:, 1], feat2[:, 2]),
        _pack_pair(feat2[:, 3], feat2[:, 4]),
        _pack_pair(feat2[:, 5], feat2[:, 6]),
        _pack_pair(feat2[:, 7], zero),
    ], axis=1)
    pairs2d = pairs.reshape(B * NPAIR * DHW // 8, 8)

    sampler = _make_sc_sampler(B, NR, P, D, H, W, CF)
    dens_flat, feat_flat = sampler(pairs2d, ox, oy, oz, dx, dy, dz, tflat)
    rd = dens_flat.reshape(B, NR, P, 1)
    rf = feat_flat.reshape(B, NR, P, CF)
    return (rd, rf)


# speedup vs baseline: 8.2688x; 8.2688x over previous
import functools

import jax
import jax.numpy as jnp
from jax import lax
from jax.experimental import pallas as pl
from jax.experimental.pallas import tpu as pltpu
from jax.experimental.pallas import tpu_sc as plsc

NC, NS, L = 2, 16, 16
NW = NC * NS
NPAIR = 5


def _make_sc_sampler(B, NR, P, D, H, W, CF):
    N = B * NR * P
    NRAYS = B * NR
    RPT = NRAYS // NW
    PPT = RPT * P
    CP = 64
    GROUPS = CP // L
    RAYS_PER_CHUNK = CP // P
    GROUPS_PER_RAY = P // L
    CHUNKS = PPT // CP
    CC = 1 + CF
    DHW = D * H * W
    DHW8 = DHW // 8
    NSLOT = 8 * NPAIR
    assert NRAYS % NW == 0 and P % L == 0 and CP == P and PPT % CP == 0
    assert CF == 8 and W % 8 == 0

    mesh = plsc.VectorSubcoreMesh(core_axis_name="c", subcore_axis_name="s")

    @functools.partial(
        pl.kernel,
        mesh=mesh,
        compiler_params=pltpu.CompilerParams(
            needs_layout_passes=False, use_tc_tiling_on_sc=False),
        out_type=(
            jax.ShapeDtypeStruct((B, NR, P), jnp.float32),
            jax.ShapeDtypeStruct((B, NR, P, CF), jnp.float32),
        ),
        scratch_types=[
            pltpu.VMEM((RPT,), jnp.float32),
            pltpu.VMEM((RPT,), jnp.float32),
            pltpu.VMEM((RPT,), jnp.float32),
            pltpu.VMEM((RPT,), jnp.float32),
            pltpu.VMEM((RPT,), jnp.float32),
            pltpu.VMEM((RPT,), jnp.float32),
            pltpu.VMEM((CP,), jnp.float32),
            pltpu.VMEM((NSLOT, CP), jnp.int32),
            pltpu.VMEM((8, GROUPS, L), jnp.float32),
            pltpu.VMEM((2, GROUPS, L), jnp.int32),
            pltpu.VMEM((NSLOT, CP, 8), jnp.int32),
            pltpu.VMEM((CP,), jnp.float32),
            pltpu.VMEM((CP, CF), jnp.float32),
            pltpu.SemaphoreType.DMA,
        ],
    )
    def sampler(pairs_h, ox_h, oy_h, oz_h, dx_h, dy_h, dz_h, t_h,
                dens_h, feat_h,
                oxv, oyv, ozv, dxv, dyv, dzv, tv, idxv, wv, ov, rowsv,
                densv, featv, gsem):
        cid = lax.axis_index("c")
        sid = lax.axis_index("s")
        wid = sid * NC + cid
        ray_base = wid * RPT
        pt_base = wid * PPT
        pair_base = (ray_base // NR) * (NPAIR * DHW8)

        pltpu.sync_copy(ox_h.at[pl.ds(ray_base, RPT)], oxv)
        pltpu.sync_copy(oy_h.at[pl.ds(ray_base, RPT)], oyv)
        pltpu.sync_copy(oz_h.at[pl.ds(ray_base, RPT)], ozv)
        pltpu.sync_copy(dx_h.at[pl.ds(ray_base, RPT)], dxv)
        pltpu.sync_copy(dy_h.at[pl.ds(ray_base, RPT)], dyv)
        pltpu.sync_copy(dz_h.at[pl.ds(ray_base, RPT)], dzv)

        iota = lax.iota(jnp.int32, L)
        fone = jnp.full((L,), 1.0, jnp.float32)
        fzero = jnp.full((L,), 0.0, jnp.float32)

        def axis_setup(pval, extent):
            gc = (pval + 1.0) * (0.5 * (extent - 1))
            ti = gc.astype(jnp.int32)
            tf = ti.astype(jnp.float32)
            neg = (gc < tf)
            lo = ti - neg.astype(jnp.int32)
            lof = tf - neg.astype(jnp.float32)
            fr = gc - lof
            w_lo = fone - fr
            w_hi = fr
            v_lo = (lo >= 0) & (lo <= extent - 1)
            v_hi = (lo >= -1) & (lo <= extent - 2)
            w_lo = jnp.where(v_lo, w_lo, fzero)
            w_hi = jnp.where(v_hi, w_hi, fzero)
            lo_c = jnp.clip(lo, 0, extent - 1)
            hi_c = jnp.clip(lo + 1, 0, extent - 1)
            return w_lo, w_hi, lo_c, hi_c

        def chunk_body(ci, carry):
            ray_g = ray_base + ci
            bl = ray_g // NR
            rl = ray_g % NR
            pltpu.sync_copy(t_h.at[bl, rl], tv)
            for g in range(GROUPS):
                ray_l = ci * RAYS_PER_CHUNK + (g // GROUPS_PER_RAY)
                ridx = jnp.full((L,), ray_l, jnp.int32)
                oxs = plsc.load_gather(oxv, [ridx])
                oys = plsc.load_gather(oyv, [ridx])
                ozs = plsc.load_gather(ozv, [ridx])
                dxs = plsc.load_gather(dxv, [ridx])
                dys = plsc.load_gather(dyv, [ridx])
                dzs = plsc.load_gather(dzv, [ridx])
                t16 = tv[pl.ds(g * L, L)]
                px = oxs + dxs * t16
                py = oys + dys * t16
                pz = ozs + dzs * t16
                wx0, wx1, x0, x1 = axis_setup(px, W)
                wy0, wy1, y0, y1 = axis_setup(py, H)
                wz0, wz1, z0, z1 = axis_setup(pz, D)
                xr0 = jnp.right_shift(x0, 3)
                xr1 = jnp.right_shift(x1, 3)
                ov[0, g, :] = jnp.bitwise_and(x0, 7)
                ov[1, g, :] = jnp.bitwise_and(x1, 7)
                zy = (
                    z0 * (H * W // 8) + y0 * (W // 8),
                    z0 * (H * W // 8) + y1 * (W // 8),
                    z1 * (H * W // 8) + y0 * (W // 8),
                    z1 * (H * W // 8) + y1 * (W // 8),
                )
                wzy = (wy0 * wz0, wy1 * wz0, wy0 * wz1, wy1 * wz1)
                for j in range(4):
                    for xi, (xr, wx) in enumerate(((xr0, wx0), (xr1, wx1))):
                        kw = j * 2 + xi
                        wv[kw, g, :] = wzy[j] * wx
                        base = zy[j] + xr + pair_base
                        for w in range(NPAIR):
                            idxv[kw * NPAIR + w, pl.ds(g * L, L)] = (
                                base + w * DHW8)
            handles = [
                pltpu.async_copy(pairs_h.at[idxv.at[s]], rowsv.at[s], gsem)
                for s in range(NSLOT)
            ]
            for h in handles:
                h.wait()
            for g in range(GROUPS):
                p_idx = iota + (g * L)
                accs = [None] * (CC + 1)

                def acc_add(c, contrib):
                    accs[c] = contrib if accs[c] is None else accs[c] + contrib

                o0v = ov[0, g, :]
                o1v = ov[1, g, :]
                for kw in range(8):
                    wk = wv[kw, g, :]
                    ovv = o1v if (kw & 1) else o0v
                    for w in range(NPAIR):
                        slot = kw * NPAIR + w
                        svec = jnp.full((L,), slot, jnp.int32)
                        val = plsc.load_gather(rowsv, [svec, p_idx, ovv])
                        vb = plsc.bitcast(val, jnp.bfloat16)
                        alo, ahi = plsc.unpack(
                            vb, format=plsc.PackFormat.INTERLEAVED)
                        acc_add(2 * w, wk * alo)
                        if 2 * w + 1 < CC:
                            acc_add(2 * w + 1, wk * ahi)
                densv[pl.ds(g * L, L)] = accs[0]
                for c in range(1, CC):
                    plsc.store_scatter(
                        featv, [p_idx, jnp.full((L,), c - 1, jnp.int32)],
                        accs[c])
            pltpu.sync_copy(densv, dens_h.at[bl, rl])
            pltpu.sync_copy(featv, feat_h.at[bl, rl])
            return carry

        lax.fori_loop(0, CHUNKS, chunk_body, 0)

    return sampler


def _bf16_bits(x):
    u = lax.bitcast_convert_type(x, jnp.uint32)
    return (u + 0x7FFF + (jnp.right_shift(u, 16) & 1)) >> 16


def _pack_pair(lo, hi):
    word = _bf16_bits(lo) | (_bf16_bits(hi) << 16)
    return lax.bitcast_convert_type(word, jnp.int32)


def kernel(origins, directions, lengths, densities, features, world2local):
    B, NR, _ = origins.shape
    P = lengths.shape[-1]
    _, CD, D, H, W = densities.shape
    CF = features.shape[1]
    DHW = D * H * W

    ones = jnp.ones(origins.shape[:-1] + (1,), dtype=origins.dtype)
    o_h = jnp.concatenate([origins, ones], axis=-1)
    o_loc = jnp.einsum('bnk,bkj->bnj', o_h, world2local)
    o_loc = o_loc[..., :3] / o_loc[..., 3:4]
    d_loc = jnp.einsum('bnk,bkj->bnj', directions, world2local[:, :3, :3])

    ox = o_loc[..., 0].reshape(-1)
    oy = o_loc[..., 1].reshape(-1)
    oz = o_loc[..., 2].reshape(-1)
    dx = d_loc[..., 0].reshape(-1)
    dy = d_loc[..., 1].reshape(-1)
    dz = d_loc[..., 2].reshape(-1)

    d0 = densities[:, 0]
    zero = jnp.zeros_like(d0)
    pairs = jnp.stack([
        _pack_pair(d0, features[:, 0]),
        _pack_pair(features[:, 1], features[:, 2]),
        _pack_pair(features[:, 3], features[:, 4]),
        _pack_pair(features[:, 5], features[:, 6]),
        _pack_pair(features[:, 7], zero),
    ], axis=1)
    pairs2d = pairs.reshape(B * NPAIR * DHW // 8, 8)

    sampler = _make_sc_sampler(B, NR, P, D, H, W, CF)
    dens_out, feat_out = sampler(pairs2d, ox, oy, oz, dx, dy, dz, lengths)
    return (dens_out[..., None], feat_out)
